# Initial kernel scaffold; baseline (speedup 1.0000x reference)
import jax
import jax.numpy as jnp
from jax.experimental import pallas as pl


def kernel(peptide_x, mhc_x, W_pep, W_mhc):
    raise NotImplementedError("write your pallas kernel here")


# speedup vs baseline: 3.0285x; 3.0285x over previous
import functools

import jax
import jax.numpy as jnp
import numpy as np
from jax import lax
from jax.experimental import pallas as pl
from jax.experimental.pallas import tpu as pltpu
from jax.experimental.pallas import tpu_sc as plsc

B = 4096
PEP_LEN = 21
MHC_LEN = 34
EMB = 128
VOCAB = 25
PEP_PAD = 3
MID = PEP_LEN - 2 * PEP_PAD

NUM_CORES = 2
NUM_SUBCORES = 16
NW = NUM_CORES * NUM_SUBCORES
ROWS_PER_TILE = B // NW

PEP_TOK = B * PEP_LEN
MHC_TOK = B * MHC_LEN
PEP_PER_TILE = PEP_TOK // NW
MHC_PER_TILE = MHC_TOK // NW
CHUNK = 128
PEP_CHUNKS = PEP_PER_TILE // CHUNK
MHC_CHUNKS = MHC_PER_TILE // CHUNK

N_GROUPS = PEP_LEN + MHC_LEN
GSTRIDE = 32
FUSED_ROWS = N_GROUPS * GSTRIDE


def _pe(max_len, d):
    pos = np.arange(max_len, dtype=np.float32)[:, None]
    div = np.exp(np.arange(0, d, 2, dtype=np.float32) * (-np.log(10000.0) / d))
    pe = np.zeros((max_len, d), dtype=np.float32)
    pe[:, 0::2] = np.sin(pos * div)
    pe[:, 1::2] = np.cos(pos * div)
    return pe


def _pe_cat():
    pe_pep = np.zeros((PEP_LEN, EMB), dtype=np.float32)
    pe_pep[PEP_PAD:PEP_LEN - PEP_PAD] = _pe(100, EMB)[:MID]
    pe_mhc = _pe(MHC_LEN, EMB)
    return jnp.asarray(np.concatenate([pe_pep, pe_mhc], axis=0))


_PE_CAT = _pe_cat()
_POS_PEP = jnp.asarray(
    (np.arange(PEP_PER_TILE, dtype=np.int32) % PEP_LEN) * GSTRIDE)
_POS_MHC = jnp.asarray(
    (PEP_LEN + np.arange(MHC_PER_TILE, dtype=np.int32) % MHC_LEN) * GSTRIDE)


def _sc_body(pep_idx, mhc_idx, wcat, pecat, pos_pep, pos_mhc,
             fused, pep_out, mhc_out, mask_out,
             w_v, pe_v, fb, posp_v, posm_v, idxp_v, idxm_v,
             idx2a, idx2b, bufa, bufb, maskb, sem_a, sem_b, sem_w):
    c = lax.axis_index("c")
    s = lax.axis_index("s")
    wid = s * NUM_CORES + c

    pltpu.sync_copy(pep_idx.at[pl.ds(wid * PEP_PER_TILE, PEP_PER_TILE)], idxp_v)
    pltpu.sync_copy(mhc_idx.at[pl.ds(wid * MHC_PER_TILE, MHC_PER_TILE)], idxm_v)
    pltpu.sync_copy(wcat, w_v)
    pltpu.sync_copy(pecat, pe_v)
    pltpu.sync_copy(pos_pep, posp_v)
    pltpu.sync_copy(pos_mhc, posm_v)

    def mask_row(r, carry):
        vals = idxp_v[pl.ds(r * PEP_LEN + PEP_PAD, 16)]
        maskb[r, :] = jnp.where(vals != 0, 1, 0).astype(jnp.int32)
        return carry

    lax.fori_loop(0, ROWS_PER_TILE, mask_row, 0)
    pltpu.sync_copy(maskb, mask_out.at[pl.ds(wid * ROWS_PER_TILE, ROWS_PER_TILE)])

    for j in range((N_GROUPS + NUM_SUBCORES - 1) // NUM_SUBCORES):
        g = s + NUM_SUBCORES * j

        @pl.when(g < N_GROUPS)
        def _():
            wbase = jnp.where(g < PEP_LEN, 0, VOCAB)
            for v in range(VOCAB):
                for k in range(EMB // 16):
                    fb[v, pl.ds(16 * k, 16)] = (
                        w_v[wbase + v, pl.ds(16 * k, 16)]
                        + pe_v[g, pl.ds(16 * k, 16)])
            pltpu.sync_copy(
                fb, fused.at[pl.ds(c * FUSED_ROWS + g * GSTRIDE, GSTRIDE), :])

    plsc.subcore_barrier()

    cbase = c * FUSED_ROWS

    def make_idx(idx_v, pos_v, idx2, t):
        off = t * CHUNK
        for k in range(CHUNK // 16):
            idx2[pl.ds(16 * k, 16)] = (
                idx_v[pl.ds(off + 16 * k, 16)]
                + pos_v[pl.ds(off + 16 * k, 16)] + cbase)

    def run_table(idx_v, pos_v, out_ref, out_base, nchunks):
        make_idx(idx_v, pos_v, idx2a, 0)
        cp = pltpu.async_copy(fused.at[idx2a], bufa, sem_a)

        def step(t, carry):
            even = lax.rem(t, 2) == 0

            @pl.when(even)
            def _():
                @pl.when(t + 1 < nchunks)
                def _():
                    make_idx(idx_v, pos_v, idx2b, t + 1)
                    pltpu.async_copy(fused.at[idx2b], bufb, sem_b)
                pltpu.make_async_copy(fused.at[idx2a], bufa, sem_a).wait()
                pltpu.async_copy(
                    bufa, out_ref.at[pl.ds(out_base + t * CHUNK, CHUNK)], sem_w
                ).wait()

            @pl.when(jnp.logical_not(even))
            def _():
                @pl.when(t + 1 < nchunks)
                def _():
                    make_idx(idx_v, pos_v, idx2a, t + 1)
                    pltpu.async_copy(fused.at[idx2a], bufa, sem_a)
                pltpu.make_async_copy(fused.at[idx2b], bufb, sem_b).wait()
                pltpu.async_copy(
                    bufb, out_ref.at[pl.ds(out_base + t * CHUNK, CHUNK)], sem_w
                ).wait()

            return carry

        lax.fori_loop(0, nchunks, step, 0)
        return cp

    run_table(idxp_v, posp_v, pep_out, wid * PEP_PER_TILE, PEP_CHUNKS)
    run_table(idxm_v, posm_v, mhc_out, wid * MHC_PER_TILE, MHC_CHUNKS)


@jax.jit
def _network_sc(pep_idx_flat, mhc_idx_flat, wcat):
    mesh = plsc.VectorSubcoreMesh(
        core_axis_name="c", subcore_axis_name="s",
        num_cores=NUM_CORES, num_subcores=NUM_SUBCORES)
    run = functools.partial(
        pl.kernel,
        out_type=[
            jax.ShapeDtypeStruct((NUM_CORES * FUSED_ROWS, EMB), jnp.float32),
            jax.ShapeDtypeStruct((PEP_TOK, EMB), jnp.float32),
            jax.ShapeDtypeStruct((MHC_TOK, EMB), jnp.float32),
            jax.ShapeDtypeStruct((B, 16), jnp.int32),
        ],
        mesh=mesh,
        scratch_types=[
            pltpu.VMEM((2 * VOCAB, EMB), jnp.float32),
            pltpu.VMEM((N_GROUPS, EMB), jnp.float32),
            pltpu.VMEM((GSTRIDE, EMB), jnp.float32),
            pltpu.VMEM((PEP_PER_TILE,), jnp.int32),
            pltpu.VMEM((MHC_PER_TILE,), jnp.int32),
            pltpu.VMEM((PEP_PER_TILE,), jnp.int32),
            pltpu.VMEM((MHC_PER_TILE,), jnp.int32),
            pltpu.VMEM((CHUNK,), jnp.int32),
            pltpu.VMEM((CHUNK,), jnp.int32),
            pltpu.VMEM((CHUNK, EMB), jnp.float32),
            pltpu.VMEM((CHUNK, EMB), jnp.float32),
            pltpu.VMEM((ROWS_PER_TILE, 16), jnp.int32),
            pltpu.SemaphoreType.DMA,
            pltpu.SemaphoreType.DMA,
            pltpu.SemaphoreType.DMA,
        ],
    )(_sc_body)
    _, pep_out, mhc_out, mask16 = run(
        pep_idx_flat, mhc_idx_flat, wcat, _PE_CAT, _POS_PEP, _POS_MHC)
    return pep_out, mhc_out, mask16


def kernel(peptide_x, mhc_x, W_pep, W_mhc):
    pep_idx = peptide_x.astype(jnp.int32).reshape(-1)
    mhc_idx = mhc_x.astype(jnp.int32).reshape(-1)
    wcat = jnp.concatenate([W_pep, W_mhc], axis=0)
    pep_out, mhc_out, mask16 = _network_sc(pep_idx, mhc_idx, wcat)
    return (
        pep_out.reshape(B, PEP_LEN, EMB),
        mhc_out.reshape(B, MHC_LEN, EMB),
        mask16[:, :MID].astype(bool),
    )


# speedup vs baseline: 4.8436x; 1.5993x over previous
import functools

import jax
import jax.numpy as jnp
import numpy as np
from jax import lax
from jax.experimental import pallas as pl
from jax.experimental.pallas import tpu as pltpu
from jax.experimental.pallas import tpu_sc as plsc

B = 4096
PEP_LEN = 21
MHC_LEN = 34
EMB = 128
VOCAB = 25
PEP_PAD = 3
MID = PEP_LEN - 2 * PEP_PAD

NUM_CORES = 2
NUM_SUBCORES = 16
NW = NUM_CORES * NUM_SUBCORES
ROWS_PER_TILE = B // NW

PEP_TOK = B * PEP_LEN
MHC_TOK = B * MHC_LEN
PEP_PER_TILE = PEP_TOK // NW
MHC_PER_TILE = MHC_TOK // NW
KP = 16
KM = 8
TP = KP * PEP_LEN
TM = KM * MHC_LEN
NP = ROWS_PER_TILE // KP
NM = ROWS_PER_TILE // KM

N_GROUPS = PEP_LEN + MHC_LEN
GSTRIDE = 32
FUSED_ROWS = N_GROUPS * GSTRIDE


def _pe(max_len, d):
    pos = np.arange(max_len, dtype=np.float32)[:, None]
    div = np.exp(np.arange(0, d, 2, dtype=np.float32) * (-np.log(10000.0) / d))
    pe = np.zeros((max_len, d), dtype=np.float32)
    pe[:, 0::2] = np.sin(pos * div)
    pe[:, 1::2] = np.cos(pos * div)
    return pe


def _pe_cat():
    pe_pep = np.zeros((PEP_LEN, EMB), dtype=np.float32)
    pe_pep[PEP_PAD:PEP_LEN - PEP_PAD] = _pe(100, EMB)[:MID]
    pe_mhc = _pe(MHC_LEN, EMB)
    return jnp.asarray(np.concatenate([pe_pep, pe_mhc], axis=0))


_PE_CAT = _pe_cat()
_POS_PEP = jnp.asarray(
    (np.arange(PEP_PER_TILE, dtype=np.int32) % PEP_LEN) * GSTRIDE)
_POS_MHC = jnp.asarray(
    (PEP_LEN + np.arange(MHC_PER_TILE, dtype=np.int32) % MHC_LEN) * GSTRIDE)


def _sc_body(pep_idx, mhc_idx, wcat, pecat, pos_pep, pos_mhc,
             fused, pep_out, mhc_out, mask_out,
             w_v, pe_v, fb, posp_v, posm_v, idxp_v, idxm_v,
             idx2p, idx2m, bufp, bufm, maskb,
             sem_p, sem_m, sem_wp, sem_wm):
    c = lax.axis_index("c")
    s = lax.axis_index("s")
    wid = s * NUM_CORES + c

    pltpu.sync_copy(pep_idx.at[pl.ds(wid * PEP_PER_TILE, PEP_PER_TILE)], idxp_v)
    pltpu.sync_copy(mhc_idx.at[pl.ds(wid * MHC_PER_TILE, MHC_PER_TILE)], idxm_v)
    pltpu.sync_copy(wcat, w_v)
    pltpu.sync_copy(pecat, pe_v)
    pltpu.sync_copy(pos_pep, posp_v)
    pltpu.sync_copy(pos_mhc, posm_v)

    def mask_row(r, carry):
        vals = idxp_v[pl.ds(r * PEP_LEN + PEP_PAD, 16)]
        maskb[r, :] = jnp.where(vals != 0, 1, 0).astype(jnp.int32)
        return carry

    lax.fori_loop(0, ROWS_PER_TILE, mask_row, 0)
    pltpu.sync_copy(maskb, mask_out.at[pl.ds(wid * ROWS_PER_TILE, ROWS_PER_TILE)])

    for j in range((N_GROUPS + NUM_SUBCORES - 1) // NUM_SUBCORES):
        g = s + NUM_SUBCORES * j

        @pl.when(g < N_GROUPS)
        def _():
            wbase = jnp.where(g < PEP_LEN, 0, VOCAB)

            def fuse_row(v, carry):
                for k in range(EMB // 16):
                    fb[v, pl.ds(16 * k, 16)] = (
                        w_v[wbase + v, pl.ds(16 * k, 16)]
                        + pe_v[g, pl.ds(16 * k, 16)])
                return carry

            lax.fori_loop(0, VOCAB, fuse_row, 0)
            pltpu.sync_copy(
                fb, fused.at[pl.ds(c * FUSED_ROWS + g * GSTRIDE, GSTRIDE), :])

    plsc.subcore_barrier()

    cbase = c * FUSED_ROWS
    row0 = wid * ROWS_PER_TILE

    def make_idx(idx_v, pos_v, idx2, ntok, t):
        off = t * ntok
        for k in range(ntok // 16):
            idx2[pl.ds(16 * k, 16)] = (
                idx_v[pl.ds(off + 16 * k, 16)]
                + pos_v[pl.ds(off + 16 * k, 16)] + cbase)

    idxrefp = idx2p
    idxrefm = idx2m
    wbufp = bufp.reshape(KP, PEP_LEN, EMB)
    wbufm = bufm.reshape(KM, MHC_LEN, EMB)

    def pep_dst(j):
        return pep_out.at[pl.ds(row0 + j * KP, KP)]

    def mhc_dst(j):
        return mhc_out.at[pl.ds(row0 + j * KM, KM)]

    make_idx(idxp_v, posp_v, idx2p, TP, 0)
    pltpu.async_copy(fused.at[idxrefp], bufp, sem_p)
    make_idx(idxm_v, posm_v, idx2m, TM, 0)
    pltpu.async_copy(fused.at[idxrefm], bufm, sem_m)

    for j in range(NM):
        if j < NP:
            pltpu.make_async_copy(fused.at[idxrefp], bufp, sem_p).wait()
            pltpu.async_copy(wbufp, pep_dst(j), sem_wp)
        pltpu.make_async_copy(fused.at[idxrefm], bufm, sem_m).wait()
        pltpu.async_copy(wbufm, mhc_dst(j), sem_wm)
        if j + 1 < NP:
            pltpu.make_async_copy(wbufp, pep_dst(j), sem_wp).wait()
            make_idx(idxp_v, posp_v, idx2p, TP, j + 1)
            pltpu.async_copy(fused.at[idxrefp], bufp, sem_p)
        if j + 1 < NM:
            pltpu.make_async_copy(wbufm, mhc_dst(j), sem_wm).wait()
            make_idx(idxm_v, posm_v, idx2m, TM, j + 1)
            pltpu.async_copy(fused.at[idxrefm], bufm, sem_m)

    pltpu.make_async_copy(wbufp, pep_dst(NP - 1), sem_wp).wait()
    pltpu.make_async_copy(wbufm, mhc_dst(NM - 1), sem_wm).wait()


@jax.jit
def _network_sc(pep_idx_flat, mhc_idx_flat, wcat):
    mesh = plsc.VectorSubcoreMesh(
        core_axis_name="c", subcore_axis_name="s",
        num_cores=NUM_CORES, num_subcores=NUM_SUBCORES)
    run = functools.partial(
        pl.kernel,
        out_type=[
            jax.ShapeDtypeStruct((NUM_CORES * FUSED_ROWS, EMB), jnp.float32),
            jax.ShapeDtypeStruct((B, PEP_LEN, EMB), jnp.float32),
            jax.ShapeDtypeStruct((B, MHC_LEN, EMB), jnp.float32),
            jax.ShapeDtypeStruct((B, 16), jnp.int32),
        ],
        mesh=mesh,
        scratch_types=[
            pltpu.VMEM((2 * VOCAB, EMB), jnp.float32),
            pltpu.VMEM((N_GROUPS, EMB), jnp.float32),
            pltpu.VMEM((GSTRIDE, EMB), jnp.float32),
            pltpu.VMEM((PEP_PER_TILE,), jnp.int32),
            pltpu.VMEM((MHC_PER_TILE,), jnp.int32),
            pltpu.VMEM((PEP_PER_TILE,), jnp.int32),
            pltpu.VMEM((MHC_PER_TILE,), jnp.int32),
            pltpu.VMEM((TP,), jnp.int32),
            pltpu.VMEM((TM,), jnp.int32),
            pltpu.VMEM((TP, EMB), jnp.float32),
            pltpu.VMEM((TM, EMB), jnp.float32),
            pltpu.VMEM((ROWS_PER_TILE, 16), jnp.int32),
            pltpu.SemaphoreType.DMA,
            pltpu.SemaphoreType.DMA,
            pltpu.SemaphoreType.DMA,
            pltpu.SemaphoreType.DMA,
        ],
    )(_sc_body)
    _, pep_out, mhc_out, mask16 = run(
        pep_idx_flat, mhc_idx_flat, wcat, _PE_CAT, _POS_PEP, _POS_MHC)
    return pep_out, mhc_out, mask16


def kernel(peptide_x, mhc_x, W_pep, W_mhc):
    pep_idx = peptide_x.astype(jnp.int32).reshape(-1)
    mhc_idx = mhc_x.astype(jnp.int32).reshape(-1)
    wcat = jnp.concatenate([W_pep, W_mhc], axis=0)
    pep_out, mhc_out, mask16 = _network_sc(pep_idx, mhc_idx, wcat)
    return (pep_out, mhc_out, mask16[:, :MID].astype(bool))


# speedup vs baseline: 5.0094x; 1.0342x over previous
import functools

import jax
import jax.numpy as jnp
import numpy as np
from jax import lax
from jax.experimental import pallas as pl
from jax.experimental.pallas import tpu as pltpu
from jax.experimental.pallas import tpu_sc as plsc

B = 4096
PEP_LEN = 21
MHC_LEN = 34
EMB = 128
VOCAB = 25
PEP_PAD = 3
MID = PEP_LEN - 2 * PEP_PAD

NUM_CORES = 2
NUM_SUBCORES = 16
NW = NUM_CORES * NUM_SUBCORES
ROWS_PER_TILE = B // NW

PEP_TOK = B * PEP_LEN
MHC_TOK = B * MHC_LEN
PEP_PER_TILE = PEP_TOK // NW
MHC_PER_TILE = MHC_TOK // NW
KP = 16
KM = 8
TP = KP * PEP_LEN
TM = KM * MHC_LEN
NP = ROWS_PER_TILE // KP
NM = ROWS_PER_TILE // KM

N_GROUPS = PEP_LEN + MHC_LEN
GSTRIDE = 32
FUSED_ROWS = N_GROUPS * GSTRIDE


def _pe(max_len, d):
    pos = np.arange(max_len, dtype=np.float32)[:, None]
    div = np.exp(np.arange(0, d, 2, dtype=np.float32) * (-np.log(10000.0) / d))
    pe = np.zeros((max_len, d), dtype=np.float32)
    pe[:, 0::2] = np.sin(pos * div)
    pe[:, 1::2] = np.cos(pos * div)
    return pe


def _pe_cat():
    pe_pep = np.zeros((PEP_LEN, EMB), dtype=np.float32)
    pe_pep[PEP_PAD:PEP_LEN - PEP_PAD] = _pe(100, EMB)[:MID]
    pe_mhc = _pe(MHC_LEN, EMB)
    return jnp.asarray(np.concatenate([pe_pep, pe_mhc], axis=0))


_PE_CAT = _pe_cat()


def _sc_body(pep_flat, pepT, mhcT, wcat, pecat,
             fused, pepT_out, mhcT_out, mask_out,
             w_v, pe_v, fb, idxp_v, idxT_p, idxT_m,
             idx2a, idx2b, bufa, bufb, maskb,
             sem_a, sem_b, sem_wa, sem_wb):
    c = lax.axis_index("c")
    s = lax.axis_index("s")
    wid = s * NUM_CORES + c
    row0 = wid * ROWS_PER_TILE

    pltpu.sync_copy(pep_flat.at[pl.ds(wid * PEP_PER_TILE, PEP_PER_TILE)],
                    idxp_v)
    pltpu.sync_copy(pepT.at[:, pl.ds(row0, ROWS_PER_TILE)], idxT_p)
    pltpu.sync_copy(mhcT.at[:, pl.ds(row0, ROWS_PER_TILE)], idxT_m)
    pltpu.sync_copy(wcat, w_v)
    pltpu.sync_copy(pecat, pe_v)

    def mask_row(r, carry):
        vals = idxp_v[pl.ds(r * PEP_LEN + PEP_PAD, 16)]
        maskb[r, :] = jnp.where(vals != 0, 1, 0).astype(jnp.int32)
        return carry

    lax.fori_loop(0, ROWS_PER_TILE, mask_row, 0)
    pltpu.sync_copy(maskb,
                    mask_out.at[pl.ds(wid * ROWS_PER_TILE, ROWS_PER_TILE)])

    for j in range((N_GROUPS + NUM_SUBCORES - 1) // NUM_SUBCORES):
        g = s + NUM_SUBCORES * j

        @pl.when(g < N_GROUPS)
        def _():
            wbase = jnp.where(g < PEP_LEN, 0, VOCAB)

            def fuse_row(v, carry):
                for k in range(EMB // 16):
                    fb[v, pl.ds(16 * k, 16)] = (
                        w_v[wbase + v, pl.ds(16 * k, 16)]
                        + pe_v[g, pl.ds(16 * k, 16)])
                return carry

            lax.fori_loop(0, VOCAB, fuse_row, 0)
            pltpu.sync_copy(
                fb, fused.at[pl.ds(c * FUSED_ROWS + g * GSTRIDE, GSTRIDE), :])

    plsc.subcore_barrier()

    cbase = c * FUSED_ROWS
    NCHUNKS = N_GROUPS

    def idx_row(i):
        return idxT_p.at[i] if i < PEP_LEN else idxT_m.at[i - PEP_LEN]

    def dst(i):
        if i < PEP_LEN:
            return pepT_out.at[i, pl.ds(row0, ROWS_PER_TILE)]
        return mhcT_out.at[i - PEP_LEN, pl.ds(row0, ROWS_PER_TILE)]

    def make_idx(i, idx2):
        src_row = idx_row(i)
        base = cbase + i * GSTRIDE
        for k in range(ROWS_PER_TILE // 16):
            idx2[pl.ds(16 * k, 16)] = src_row[pl.ds(16 * k, 16)] + base

    bufs = (bufa, bufb)
    idx2s = (idx2a, idx2b)
    gsems = (sem_a, sem_b)
    wsems = (sem_wa, sem_wb)

    make_idx(0, idx2a)
    pltpu.async_copy(fused.at[idx2a], bufa, sem_a)
    for i in range(NCHUNKS):
        p, q = i % 2, (i + 1) % 2
        if i + 1 < NCHUNKS:
            if i >= 1:
                pltpu.make_async_copy(bufs[q], dst(i - 1), wsems[q]).wait()
            make_idx(i + 1, idx2s[q])
            pltpu.async_copy(fused.at[idx2s[q]], bufs[q], gsems[q])
        pltpu.make_async_copy(fused.at[idx2s[p]], bufs[p], gsems[p]).wait()
        pltpu.async_copy(bufs[p], dst(i), wsems[p])

    pltpu.make_async_copy(bufs[(NCHUNKS - 1) % 2], dst(NCHUNKS - 1),
                          wsems[(NCHUNKS - 1) % 2]).wait()
    pltpu.make_async_copy(bufs[(NCHUNKS - 2) % 2], dst(NCHUNKS - 2),
                          wsems[(NCHUNKS - 2) % 2]).wait()


@jax.jit
def _network_sc(pep_flat, pepT, mhcT, wcat):
    mesh = plsc.VectorSubcoreMesh(
        core_axis_name="c", subcore_axis_name="s",
        num_cores=NUM_CORES, num_subcores=NUM_SUBCORES)
    run = functools.partial(
        pl.kernel,
        out_type=[
            jax.ShapeDtypeStruct((NUM_CORES * FUSED_ROWS, EMB), jnp.float32),
            jax.ShapeDtypeStruct((PEP_LEN, B, EMB), jnp.float32),
            jax.ShapeDtypeStruct((MHC_LEN, B, EMB), jnp.float32),
            jax.ShapeDtypeStruct((B, 16), jnp.int32),
        ],
        mesh=mesh,
        scratch_types=[
            pltpu.VMEM((2 * VOCAB, EMB), jnp.float32),
            pltpu.VMEM((N_GROUPS, EMB), jnp.float32),
            pltpu.VMEM((GSTRIDE, EMB), jnp.float32),
            pltpu.VMEM((PEP_PER_TILE,), jnp.int32),
            pltpu.VMEM((PEP_LEN, ROWS_PER_TILE), jnp.int32),
            pltpu.VMEM((MHC_LEN, ROWS_PER_TILE), jnp.int32),
            pltpu.VMEM((ROWS_PER_TILE,), jnp.int32),
            pltpu.VMEM((ROWS_PER_TILE,), jnp.int32),
            pltpu.VMEM((ROWS_PER_TILE, EMB), jnp.float32),
            pltpu.VMEM((ROWS_PER_TILE, EMB), jnp.float32),
            pltpu.VMEM((ROWS_PER_TILE, 16), jnp.int32),
            pltpu.SemaphoreType.DMA,
            pltpu.SemaphoreType.DMA,
            pltpu.SemaphoreType.DMA,
            pltpu.SemaphoreType.DMA,
        ],
    )(_sc_body)
    _, pepT_out, mhcT_out, mask16 = run(pep_flat, pepT, mhcT, wcat, _PE_CAT)
    return pepT_out, mhcT_out, mask16


def kernel(peptide_x, mhc_x, W_pep, W_mhc):
    pep2d = peptide_x.astype(jnp.int32)
    mhc2d = mhc_x.astype(jnp.int32)
    wcat = jnp.concatenate([W_pep, W_mhc], axis=0)
    pepT_out, mhcT_out, mask16 = _network_sc(
        pep2d.reshape(-1), pep2d.T, mhc2d.T, wcat)
    return (
        pepT_out.transpose(1, 0, 2),
        mhcT_out.transpose(1, 0, 2),
        mask16[:, :MID].astype(bool),
    )


# speedup vs baseline: 5.2592x; 1.0499x over previous
import functools

import jax
import jax.numpy as jnp
import numpy as np
from jax import lax
from jax.experimental import pallas as pl
from jax.experimental.pallas import tpu as pltpu
from jax.experimental.pallas import tpu_sc as plsc

B = 4096
PEP_LEN = 21
MHC_LEN = 34
EMB = 128
VOCAB = 25
PEP_PAD = 3
MID = PEP_LEN - 2 * PEP_PAD

NUM_CORES = 2
NUM_SUBCORES = 16
NW = NUM_CORES * NUM_SUBCORES
ROWS_PER_TILE = B // NW

PEP_TOK = B * PEP_LEN
MHC_TOK = B * MHC_LEN
PEP_PER_TILE = PEP_TOK // NW
MHC_PER_TILE = MHC_TOK // NW
KP = 16
KM = 8
TP = KP * PEP_LEN
TM = KM * MHC_LEN
NP = ROWS_PER_TILE // KP
NM = ROWS_PER_TILE // KM

N_GROUPS = PEP_LEN + MHC_LEN
GSTRIDE = 32
FUSED_ROWS = N_GROUPS * GSTRIDE


def _pe(max_len, d):
    pos = np.arange(max_len, dtype=np.float32)[:, None]
    div = np.exp(np.arange(0, d, 2, dtype=np.float32) * (-np.log(10000.0) / d))
    pe = np.zeros((max_len, d), dtype=np.float32)
    pe[:, 0::2] = np.sin(pos * div)
    pe[:, 1::2] = np.cos(pos * div)
    return pe


def _pe_cat():
    pe_pep = np.zeros((PEP_LEN, EMB), dtype=np.float32)
    pe_pep[PEP_PAD:PEP_LEN - PEP_PAD] = _pe(100, EMB)[:MID]
    pe_mhc = _pe(MHC_LEN, EMB)
    return jnp.asarray(np.concatenate([pe_pep, pe_mhc], axis=0))


_PE_CAT = _pe_cat()


def _sc_body(pep_flat, pepT, mhcT, wcat, pecat,
             fused, pepT_out, mhcT_out, mask_out,
             w_v, pe_v, fb, idxp_v, idxT_p, idxT_m,
             idx2a, idx2b, bufa, bufb, maskb,
             sem_a, sem_b, sem_wa, sem_wb):
    c = lax.axis_index("c")
    s = lax.axis_index("s")
    wid = s * NUM_CORES + c
    row0 = wid * ROWS_PER_TILE

    pltpu.sync_copy(pep_flat.at[pl.ds(wid * PEP_PER_TILE, PEP_PER_TILE)],
                    idxp_v)
    pltpu.sync_copy(pepT.at[:, pl.ds(row0, ROWS_PER_TILE)], idxT_p)
    pltpu.sync_copy(mhcT.at[:, pl.ds(row0, ROWS_PER_TILE)], idxT_m)
    pltpu.sync_copy(wcat, w_v)
    pltpu.sync_copy(pecat, pe_v)

    def mask_row(r, carry):
        vals = idxp_v[pl.ds(r * PEP_LEN + PEP_PAD, 16)]
        maskb[r, :] = jnp.where(vals != 0, 1, 0).astype(jnp.int32)
        return carry

    lax.fori_loop(0, ROWS_PER_TILE, mask_row, 0)
    pltpu.sync_copy(maskb,
                    mask_out.at[pl.ds(wid * ROWS_PER_TILE, ROWS_PER_TILE)])

    for j in range((N_GROUPS + NUM_SUBCORES - 1) // NUM_SUBCORES):
        g = s + NUM_SUBCORES * j

        @pl.when(g < N_GROUPS)
        def _():
            wbase = jnp.where(g < PEP_LEN, 0, VOCAB)

            def fuse_row(v, carry):
                for k in range(EMB // 16):
                    fb[v, pl.ds(16 * k, 16)] = (
                        w_v[wbase + v, pl.ds(16 * k, 16)]
                        + pe_v[g, pl.ds(16 * k, 16)])
                return carry

            lax.fori_loop(0, VOCAB, fuse_row, 0)
            pltpu.sync_copy(
                fb, fused.at[pl.ds(c * FUSED_ROWS + g * GSTRIDE, GSTRIDE), :])

    plsc.subcore_barrier()

    cbase = c * FUSED_ROWS

    def idx_row(i):
        return idxT_p.at[i] if i < PEP_LEN else idxT_m.at[i - PEP_LEN]

    chunks = [(i, 2) for i in range(0, PEP_LEN - 1, 2)]
    chunks += [(i, 2) for i in range(PEP_LEN, N_GROUPS, 2)]
    tail = (PEP_LEN - 1, 1)

    R = ROWS_PER_TILE

    def dst(i0, ni):
        if i0 + ni <= PEP_LEN:
            return pepT_out.at[pl.ds(i0, ni), pl.ds(row0, R), :]
        return mhcT_out.at[pl.ds(i0 - PEP_LEN, ni), pl.ds(row0, R), :]

    def make_idx(i0, ni, idx2):
        for j in range(ni):
            row = idx_row(i0 + j)
            base = cbase + (i0 + j) * GSTRIDE
            for k in range(R // 16):
                idx2[pl.ds(j * R + 16 * k, 16)] = (
                    row[pl.ds(16 * k, 16)] + base)

    bufs = (bufa.reshape(2, R, EMB), bufb.reshape(2, R, EMB))
    flat = (bufa, bufb)
    idx2s = (idx2a, idx2b)
    gsems = (sem_a, sem_b)
    wsems = (sem_wa, sem_wb)
    NC2 = len(chunks)

    make_idx(*chunks[0], idx2a)
    pltpu.async_copy(fused.at[idx2a], flat[0], sem_a)
    for i in range(NC2):
        p, q = i % 2, (i + 1) % 2
        if i + 1 < NC2:
            if i >= 1:
                pltpu.make_async_copy(bufs[q], dst(*chunks[i - 1]),
                                      wsems[q]).wait()
            make_idx(*chunks[i + 1], idx2s[q])
            pltpu.async_copy(fused.at[idx2s[q]], flat[q], gsems[q])
        pltpu.make_async_copy(fused.at[idx2s[p]], flat[p], gsems[p]).wait()
        pltpu.async_copy(bufs[p], dst(*chunks[i]), wsems[p])

    pltpu.make_async_copy(bufs[(NC2 - 1) % 2], dst(*chunks[NC2 - 1]),
                          wsems[(NC2 - 1) % 2]).wait()
    pltpu.make_async_copy(bufs[(NC2 - 2) % 2], dst(*chunks[NC2 - 2]),
                          wsems[(NC2 - 2) % 2]).wait()

    t0, _ = tail
    row = idx_row(t0)
    base = cbase + t0 * GSTRIDE
    for k in range(R // 16):
        idx2a[pl.ds(16 * k, 16)] = row[pl.ds(16 * k, 16)] + base
    pltpu.async_copy(fused.at[idx2a.at[pl.ds(0, R)]],
                     bufa.at[pl.ds(0, R), :], sem_a).wait()
    pltpu.async_copy(bufa.at[pl.ds(0, R), :],
                     pepT_out.at[t0, pl.ds(row0, R), :], sem_wa).wait()


@jax.jit
def _network_sc(pep_flat, pepT, mhcT, wcat):
    mesh = plsc.VectorSubcoreMesh(
        core_axis_name="c", subcore_axis_name="s",
        num_cores=NUM_CORES, num_subcores=NUM_SUBCORES)
    run = functools.partial(
        pl.kernel,
        out_type=[
            jax.ShapeDtypeStruct((NUM_CORES * FUSED_ROWS, EMB), jnp.float32),
            jax.ShapeDtypeStruct((PEP_LEN, B, EMB), jnp.float32),
            jax.ShapeDtypeStruct((MHC_LEN, B, EMB), jnp.float32),
            jax.ShapeDtypeStruct((B, 16), jnp.int32),
        ],
        mesh=mesh,
        scratch_types=[
            pltpu.VMEM((2 * VOCAB, EMB), jnp.float32),
            pltpu.VMEM((N_GROUPS, EMB), jnp.float32),
            pltpu.VMEM((GSTRIDE, EMB), jnp.float32),
            pltpu.VMEM((PEP_PER_TILE,), jnp.int32),
            pltpu.VMEM((PEP_LEN, ROWS_PER_TILE), jnp.int32),
            pltpu.VMEM((MHC_LEN, ROWS_PER_TILE), jnp.int32),
            pltpu.VMEM((2 * ROWS_PER_TILE,), jnp.int32),
            pltpu.VMEM((2 * ROWS_PER_TILE,), jnp.int32),
            pltpu.VMEM((2 * ROWS_PER_TILE, EMB), jnp.float32),
            pltpu.VMEM((2 * ROWS_PER_TILE, EMB), jnp.float32),
            pltpu.VMEM((ROWS_PER_TILE, 16), jnp.int32),
            pltpu.SemaphoreType.DMA,
            pltpu.SemaphoreType.DMA,
            pltpu.SemaphoreType.DMA,
            pltpu.SemaphoreType.DMA,
        ],
    )(_sc_body)
    _, pepT_out, mhcT_out, mask16 = run(pep_flat, pepT, mhcT, wcat, _PE_CAT)
    return pepT_out, mhcT_out, mask16


def kernel(peptide_x, mhc_x, W_pep, W_mhc):
    pep2d = peptide_x.astype(jnp.int32)
    mhc2d = mhc_x.astype(jnp.int32)
    wcat = jnp.concatenate([W_pep, W_mhc], axis=0)
    pepT_out, mhcT_out, mask16 = _network_sc(
        pep2d.reshape(-1), pep2d.T, mhc2d.T, wcat)
    return (
        pepT_out.transpose(1, 0, 2),
        mhcT_out.transpose(1, 0, 2),
        mask16[:, :MID].astype(bool),
    )


# speedup vs baseline: 7.3340x; 1.3945x over previous
import functools

import jax
import jax.numpy as jnp
import numpy as np
from jax import lax
from jax.experimental import pallas as pl
from jax.experimental.pallas import tpu as pltpu
from jax.experimental.pallas import tpu_sc as plsc

B = 4096
PEP_LEN = 21
MHC_LEN = 34
EMB = 128
VOCAB = 25
PEP_PAD = 3
MID = PEP_LEN - 2 * PEP_PAD

NUM_CORES = 2
NUM_SUBCORES = 16
NW = NUM_CORES * NUM_SUBCORES
ROWS_PER_TILE = B // NW

PEP_TOK = B * PEP_LEN
MHC_TOK = B * MHC_LEN
PEP_PER_TILE = PEP_TOK // NW
MHC_PER_TILE = MHC_TOK // NW
KP = 16
KM = 8
TP = KP * PEP_LEN
TM = KM * MHC_LEN
NP = ROWS_PER_TILE // KP
NM = ROWS_PER_TILE // KM

N_GROUPS = PEP_LEN + MHC_LEN
GSTRIDE = 32
FUSED_ROWS = N_GROUPS * GSTRIDE


def _pe(max_len, d):
    pos = np.arange(max_len, dtype=np.float32)[:, None]
    div = np.exp(np.arange(0, d, 2, dtype=np.float32) * (-np.log(10000.0) / d))
    pe = np.zeros((max_len, d), dtype=np.float32)
    pe[:, 0::2] = np.sin(pos * div)
    pe[:, 1::2] = np.cos(pos * div)
    return pe


def _pe_cat():
    pe_pep = np.zeros((PEP_LEN, EMB), dtype=np.float32)
    pe_pep[PEP_PAD:PEP_LEN - PEP_PAD] = _pe(100, EMB)[:MID]
    pe_mhc = _pe(MHC_LEN, EMB)
    return jnp.asarray(np.concatenate([pe_pep, pe_mhc], axis=0))


_PE_CAT = _pe_cat()


def _sc_body(pep_flat, pepT, mhcT, wcat, pecat,
             fused, pepT_out, mhcT_out, mask_out,
             w_v, pe_v, fb, idxp_v, idxT_all,
             idx2a, idx2b, bufa, bufb, maskb,
             sem_a, sem_b, sem_wa, sem_wb):
    c = lax.axis_index("c")
    s = lax.axis_index("s")
    wid = s * NUM_CORES + c
    row0 = wid * ROWS_PER_TILE

    pltpu.sync_copy(pep_flat.at[pl.ds(wid * PEP_PER_TILE, PEP_PER_TILE)],
                    idxp_v)
    pltpu.sync_copy(pepT.at[:, pl.ds(row0, ROWS_PER_TILE)],
                    idxT_all.at[pl.ds(0, PEP_LEN), :])
    pltpu.sync_copy(mhcT.at[:, pl.ds(row0, ROWS_PER_TILE)],
                    idxT_all.at[pl.ds(PEP_LEN, MHC_LEN), :])
    pltpu.sync_copy(wcat, w_v)
    pltpu.sync_copy(pecat, pe_v)

    def mask_row(r, carry):
        vals = idxp_v[pl.ds(r * PEP_LEN + PEP_PAD, 16)]
        maskb[r, :] = jnp.where(vals != 0, 1, 0).astype(jnp.int32)
        return carry

    lax.fori_loop(0, ROWS_PER_TILE, mask_row, 0)
    pltpu.sync_copy(maskb,
                    mask_out.at[pl.ds(wid * ROWS_PER_TILE, ROWS_PER_TILE)])

    for j in range((N_GROUPS + NUM_SUBCORES - 1) // NUM_SUBCORES):
        g = s + NUM_SUBCORES * j

        @pl.when(g < N_GROUPS)
        def _():
            wbase = jnp.where(g < PEP_LEN, 0, VOCAB)

            def fuse_row(v, carry):
                for k in range(EMB // 16):
                    fb[v, pl.ds(16 * k, 16)] = (
                        w_v[wbase + v, pl.ds(16 * k, 16)]
                        + pe_v[g, pl.ds(16 * k, 16)])
                return carry

            lax.fori_loop(0, VOCAB, fuse_row, 0)
            pltpu.sync_copy(
                fb, fused.at[pl.ds(c * FUSED_ROWS + g * GSTRIDE, GSTRIDE), :])

    plsc.subcore_barrier()

    cbase = c * FUSED_ROWS
    R = ROWS_PER_TILE
    NC2 = 27
    NPP = 10

    def make_idx(i0, idx2):
        for j in range(2):
            row = idxT_all.at[i0 + j]
            base = cbase + (i0 + j) * GSTRIDE
            for k in range(R // 16):
                idx2[pl.ds(j * R + 16 * k, 16)] = (
                    row[pl.ds(16 * k, 16)] + base)

    def chunk_start(i):
        ci = lax.rem(wid + i, NC2)
        return ci, 2 * ci + jnp.where(ci >= NPP, 1, 0)

    def issue_write(ci, i0, wbuf, wsem):
        @pl.when(ci < NPP)
        def _():
            pltpu.async_copy(
                wbuf, pepT_out.at[pl.ds(i0, 2), pl.ds(row0, R), :], wsem)

        @pl.when(ci >= NPP)
        def _():
            pltpu.async_copy(
                wbuf, mhcT_out.at[pl.ds(i0 - PEP_LEN, 2), pl.ds(row0, R), :],
                wsem)

    def drain_write(wbuf, wsem):
        pltpu.make_async_copy(
            wbuf, pepT_out.at[pl.ds(0, 2), pl.ds(row0, R), :], wsem).wait()

    bufs = (bufa.reshape(2, R, EMB), bufb.reshape(2, R, EMB))
    flat = (bufa, bufb)
    idx2s = (idx2a, idx2b)
    gsems = (sem_a, sem_b)
    wsems = (sem_wa, sem_wb)

    _, i0_first = chunk_start(0)
    make_idx(i0_first, idx2a)
    pltpu.async_copy(fused.at[idx2a], flat[0], sem_a)
    for i in range(NC2):
        p, q = i % 2, (i + 1) % 2
        if i + 1 < NC2:
            if i >= 1:
                drain_write(bufs[q], wsems[q])
            _, i0n = chunk_start(i + 1)
            make_idx(i0n, idx2s[q])
            pltpu.async_copy(fused.at[idx2s[q]], flat[q], gsems[q])
        pltpu.make_async_copy(fused.at[idx2s[p]], flat[p], gsems[p]).wait()
        ci, i0 = chunk_start(i)
        issue_write(ci, i0, bufs[p], wsems[p])

    drain_write(bufs[(NC2 - 1) % 2], wsems[(NC2 - 1) % 2])
    drain_write(bufs[(NC2 - 2) % 2], wsems[(NC2 - 2) % 2])

    t0 = PEP_LEN - 1
    row = idxT_all.at[t0]
    base = cbase + t0 * GSTRIDE
    for k in range(R // 16):
        idx2a[pl.ds(16 * k, 16)] = row[pl.ds(16 * k, 16)] + base
    pltpu.async_copy(fused.at[idx2a.at[pl.ds(0, R)]],
                     bufa.at[pl.ds(0, R), :], sem_a).wait()
    pltpu.async_copy(bufa.at[pl.ds(0, R), :],
                     pepT_out.at[t0, pl.ds(row0, R), :], sem_wa).wait()


@jax.jit
def _network_sc(pep_flat, pepT, mhcT, wcat):
    mesh = plsc.VectorSubcoreMesh(
        core_axis_name="c", subcore_axis_name="s",
        num_cores=NUM_CORES, num_subcores=NUM_SUBCORES)
    run = functools.partial(
        pl.kernel,
        out_type=[
            jax.ShapeDtypeStruct((NUM_CORES * FUSED_ROWS, EMB), jnp.float32),
            jax.ShapeDtypeStruct((PEP_LEN, B, EMB), jnp.float32),
            jax.ShapeDtypeStruct((MHC_LEN, B, EMB), jnp.float32),
            jax.ShapeDtypeStruct((B, 16), jnp.int32),
        ],
        mesh=mesh,
        scratch_types=[
            pltpu.VMEM((2 * VOCAB, EMB), jnp.float32),
            pltpu.VMEM((N_GROUPS, EMB), jnp.float32),
            pltpu.VMEM((GSTRIDE, EMB), jnp.float32),
            pltpu.VMEM((PEP_PER_TILE,), jnp.int32),
            pltpu.VMEM((N_GROUPS, ROWS_PER_TILE), jnp.int32),
            pltpu.VMEM((2 * ROWS_PER_TILE,), jnp.int32),
            pltpu.VMEM((2 * ROWS_PER_TILE,), jnp.int32),
            pltpu.VMEM((2 * ROWS_PER_TILE, EMB), jnp.float32),
            pltpu.VMEM((2 * ROWS_PER_TILE, EMB), jnp.float32),
            pltpu.VMEM((ROWS_PER_TILE, 16), jnp.int32),
            pltpu.SemaphoreType.DMA,
            pltpu.SemaphoreType.DMA,
            pltpu.SemaphoreType.DMA,
            pltpu.SemaphoreType.DMA,
        ],
    )(_sc_body)
    _, pepT_out, mhcT_out, mask16 = run(pep_flat, pepT, mhcT, wcat, _PE_CAT)
    return pepT_out, mhcT_out, mask16


def kernel(peptide_x, mhc_x, W_pep, W_mhc):
    pep2d = peptide_x.astype(jnp.int32)
    mhc2d = mhc_x.astype(jnp.int32)
    wcat = jnp.concatenate([W_pep, W_mhc], axis=0)
    pepT_out, mhcT_out, mask16 = _network_sc(
        pep2d.reshape(-1), pep2d.T, mhc2d.T, wcat)
    return (
        pepT_out.transpose(1, 0, 2),
        mhcT_out.transpose(1, 0, 2),
        mask16[:, :MID].astype(bool),
    )


# speedup vs baseline: 7.3418x; 1.0011x over previous
import functools

import jax
import jax.numpy as jnp
import numpy as np
from jax import lax
from jax.experimental import pallas as pl
from jax.experimental.pallas import tpu as pltpu
from jax.experimental.pallas import tpu_sc as plsc

B = 4096
PEP_LEN = 21
MHC_LEN = 34
EMB = 128
VOCAB = 25
PEP_PAD = 3
MID = PEP_LEN - 2 * PEP_PAD

NUM_CORES = 2
NUM_SUBCORES = 16
NW = NUM_CORES * NUM_SUBCORES
ROWS_PER_TILE = B // NW

PEP_TOK = B * PEP_LEN
MHC_TOK = B * MHC_LEN
PEP_PER_TILE = PEP_TOK // NW
MHC_PER_TILE = MHC_TOK // NW
N_GROUPS = PEP_LEN + MHC_LEN
GSTRIDE = 32
FUSED_ROWS = N_GROUPS * GSTRIDE


def _pe(max_len, d):
    pos = np.arange(max_len, dtype=np.float32)[:, None]
    div = np.exp(np.arange(0, d, 2, dtype=np.float32) * (-np.log(10000.0) / d))
    pe = np.zeros((max_len, d), dtype=np.float32)
    pe[:, 0::2] = np.sin(pos * div)
    pe[:, 1::2] = np.cos(pos * div)
    return pe


def _pe_cat():
    pe_pep = np.zeros((PEP_LEN, EMB), dtype=np.float32)
    pe_pep[PEP_PAD:PEP_LEN - PEP_PAD] = _pe(100, EMB)[:MID]
    pe_mhc = _pe(MHC_LEN, EMB)
    return jnp.asarray(np.concatenate([pe_pep, pe_mhc], axis=0))


_PE_CAT = _pe_cat()


def _sc_body(pep_flat, pepT, mhcT, wcat, pecat,
             fused, pepT_out, mhcT_out, mask_out,
             w_v, pe_v, fb, idxp_v, idxT_all,
             idx2a, idx2b, bufa, bufb, maskb,
             sem_a, sem_b, sem_wa, sem_wb):
    c = lax.axis_index("c")
    s = lax.axis_index("s")
    wid = s * NUM_CORES + c
    row0 = wid * ROWS_PER_TILE

    pltpu.sync_copy(pep_flat.at[pl.ds(wid * PEP_PER_TILE, PEP_PER_TILE)],
                    idxp_v)
    pltpu.sync_copy(pepT.at[:, pl.ds(row0, ROWS_PER_TILE)],
                    idxT_all.at[pl.ds(0, PEP_LEN), :])
    pltpu.sync_copy(mhcT.at[:, pl.ds(row0, ROWS_PER_TILE)],
                    idxT_all.at[pl.ds(PEP_LEN, MHC_LEN), :])
    pltpu.sync_copy(wcat, w_v)
    pltpu.sync_copy(pecat, pe_v)

    def mask_row(r, carry):
        vals = idxp_v[pl.ds(r * PEP_LEN + PEP_PAD, 16)]
        maskb[r, :] = jnp.where(vals != 0, 1, 0).astype(jnp.int32)
        return carry

    lax.fori_loop(0, ROWS_PER_TILE, mask_row, 0)
    pltpu.sync_copy(maskb,
                    mask_out.at[pl.ds(wid * ROWS_PER_TILE, ROWS_PER_TILE)])

    for j in range((N_GROUPS + NUM_SUBCORES - 1) // NUM_SUBCORES):
        g = s + NUM_SUBCORES * j

        @pl.when(g < N_GROUPS)
        def _():
            wbase = jnp.where(g < PEP_LEN, 0, VOCAB)

            def fuse_row(v, carry):
                for k in range(EMB // 16):
                    fb[v, pl.ds(16 * k, 16)] = (
                        w_v[wbase + v, pl.ds(16 * k, 16)]
                        + pe_v[g, pl.ds(16 * k, 16)])
                return carry

            lax.fori_loop(0, VOCAB, fuse_row, 0)
            pltpu.sync_copy(
                fb, fused.at[pl.ds(c * FUSED_ROWS + g * GSTRIDE, GSTRIDE), :])

    plsc.subcore_barrier()

    cbase = c * FUSED_ROWS
    R = ROWS_PER_TILE
    NC2 = 27
    NPP = 10

    def make_idx(i0, idx2):
        for j in range(2):
            row = idxT_all.at[i0 + j]
            base = cbase + (i0 + j) * GSTRIDE
            for k in range(R // 16):
                idx2[pl.ds(j * R + 16 * k, 16)] = (
                    row[pl.ds(16 * k, 16)] + base)

    def chunk_start(i):
        ci = lax.rem(wid + i, NC2)
        return ci, 2 * ci + jnp.where(ci >= NPP, 1, 0)

    def issue_write(ci, i0, wbuf, wsem):
        @pl.when(ci < NPP)
        def _():
            pltpu.async_copy(
                wbuf, pepT_out.at[pl.ds(i0, 2), pl.ds(row0, R), :], wsem)

        @pl.when(ci >= NPP)
        def _():
            pltpu.async_copy(
                wbuf, mhcT_out.at[pl.ds(i0 - PEP_LEN, 2), pl.ds(row0, R), :],
                wsem)

    def drain_write(wbuf, wsem):
        pltpu.make_async_copy(
            wbuf, pepT_out.at[pl.ds(0, 2), pl.ds(row0, R), :], wsem).wait()

    bufs = (bufa.reshape(2, R, EMB), bufb.reshape(2, R, EMB))
    flat = (bufa, bufb)
    idx2s = (idx2a, idx2b)
    gsems = (sem_a, sem_b)
    wsems = (sem_wa, sem_wb)

    _, i0_first = chunk_start(0)
    make_idx(i0_first, idx2a)
    pltpu.async_copy(fused.at[idx2a], flat[0], sem_a)
    for i in range(NC2):
        p, q = i % 2, (i + 1) % 2
        if i + 1 < NC2:
            if i >= 1:
                drain_write(bufs[q], wsems[q])
            _, i0n = chunk_start(i + 1)
            make_idx(i0n, idx2s[q])
            pltpu.async_copy(fused.at[idx2s[q]], flat[q], gsems[q])
        pltpu.make_async_copy(fused.at[idx2s[p]], flat[p], gsems[p]).wait()
        ci, i0 = chunk_start(i)
        issue_write(ci, i0, bufs[p], wsems[p])

    drain_write(bufs[(NC2 - 1) % 2], wsems[(NC2 - 1) % 2])
    drain_write(bufs[(NC2 - 2) % 2], wsems[(NC2 - 2) % 2])

    t0 = PEP_LEN - 1
    row = idxT_all.at[t0]
    base = cbase + t0 * GSTRIDE
    for k in range(R // 16):
        idx2a[pl.ds(16 * k, 16)] = row[pl.ds(16 * k, 16)] + base
    pltpu.async_copy(fused.at[idx2a.at[pl.ds(0, R)]],
                     bufa.at[pl.ds(0, R), :], sem_a).wait()
    pltpu.async_copy(bufa.at[pl.ds(0, R), :],
                     pepT_out.at[t0, pl.ds(row0, R), :], sem_wa).wait()


@jax.jit
def _network_sc(pep_flat, pepT, mhcT, wcat):
    mesh = plsc.VectorSubcoreMesh(
        core_axis_name="c", subcore_axis_name="s",
        num_cores=NUM_CORES, num_subcores=NUM_SUBCORES)
    run = functools.partial(
        pl.kernel,
        out_type=[
            jax.ShapeDtypeStruct((NUM_CORES * FUSED_ROWS, EMB), jnp.float32),
            jax.ShapeDtypeStruct((PEP_LEN, B, EMB), jnp.float32),
            jax.ShapeDtypeStruct((MHC_LEN, B, EMB), jnp.float32),
            jax.ShapeDtypeStruct((B, 16), jnp.int32),
        ],
        mesh=mesh,
        scratch_types=[
            pltpu.VMEM((2 * VOCAB, EMB), jnp.float32),
            pltpu.VMEM((N_GROUPS, EMB), jnp.float32),
            pltpu.VMEM((GSTRIDE, EMB), jnp.float32),
            pltpu.VMEM((PEP_PER_TILE,), jnp.int32),
            pltpu.VMEM((N_GROUPS, ROWS_PER_TILE), jnp.int32),
            pltpu.VMEM((2 * ROWS_PER_TILE,), jnp.int32),
            pltpu.VMEM((2 * ROWS_PER_TILE,), jnp.int32),
            pltpu.VMEM((2 * ROWS_PER_TILE, EMB), jnp.float32),
            pltpu.VMEM((2 * ROWS_PER_TILE, EMB), jnp.float32),
            pltpu.VMEM((ROWS_PER_TILE, 16), jnp.int32),
            pltpu.SemaphoreType.DMA,
            pltpu.SemaphoreType.DMA,
            pltpu.SemaphoreType.DMA,
            pltpu.SemaphoreType.DMA,
        ],
    )(_sc_body)
    _, pepT_out, mhcT_out, mask16 = run(pep_flat, pepT, mhcT, wcat, _PE_CAT)
    return pepT_out, mhcT_out, mask16


def kernel(peptide_x, mhc_x, W_pep, W_mhc):
    pep2d = peptide_x.astype(jnp.int32)
    mhc2d = mhc_x.astype(jnp.int32)
    wcat = jnp.concatenate([W_pep, W_mhc], axis=0)
    pepT_out, mhcT_out, mask16 = _network_sc(
        pep2d.reshape(-1), pep2d.T, mhc2d.T, wcat)
    return (
        pepT_out.transpose(1, 0, 2),
        mhcT_out.transpose(1, 0, 2),
        mask16[:, :MID].astype(bool),
    )
